# baseline (speedup 1.0000x reference)
import functools

import jax
import jax.numpy as jnp
from jax import lax
from jax.experimental import pallas as pl
from jax.experimental.pallas import tpu as pltpu
from jax.experimental.pallas import tpu_sc as plsc

VOCAB = 100000
DIM = 64
NVALS = 81920

NC = 2
NS = 16
NW = NC * NS

CHUNK = 128
B_PER_W = NVALS // NW
CHUNKS_PER_W = B_PER_W // CHUNK


@functools.partial(
    pl.kernel,
    out_type=(
        jax.ShapeDtypeStruct((NVALS, DIM), jnp.float32),
        jax.ShapeDtypeStruct((NVALS, DIM), jnp.float32),
    ),
    mesh=plsc.VectorSubcoreMesh(core_axis_name="c", subcore_axis_name="s"),
    compiler_params=pltpu.CompilerParams(use_tc_tiling_on_sc=False),
    scratch_types=[
        pltpu.VMEM((CHUNKS_PER_W, CHUNK), jnp.int32),
        pltpu.VMEM((CHUNK, DIM), jnp.float32),
        pltpu.SemaphoreType.DMA,
    ],
)
def _lookup2(v1_hbm, v2_hbm, t1_hbm, t2_hbm, o1_hbm, o2_hbm,
             idx_v, rows_v, sem):
    wid = lax.axis_index("s") * NC + lax.axis_index("c")
    base = wid * B_PER_W

    for vals_hbm, table_hbm, out_hbm in ((v1_hbm, t1_hbm, o1_hbm),
                                         (v2_hbm, t2_hbm, o2_hbm)):
        pltpu.sync_copy(vals_hbm.at[wid], idx_v)

        @pl.loop(0, CHUNKS_PER_W)
        def _(j):
            pltpu.async_copy(table_hbm.at[idx_v.at[j]], rows_v, sem).wait()
            pltpu.sync_copy(rows_v, out_hbm.at[pl.ds(base + j * CHUNK, CHUNK)])


def kernel(values_f1, values_f2, table_t1, table_t2):
    v1 = values_f1.reshape(NW, CHUNKS_PER_W, CHUNK)
    v2 = values_f2.reshape(NW, CHUNKS_PER_W, CHUNK)
    return _lookup2(v1, v2, table_t1, table_t2)


# speedup vs baseline: 1.1027x; 1.1027x over previous
import functools

import jax
import jax.numpy as jnp
from jax import lax
from jax.experimental import pallas as pl
from jax.experimental.pallas import tpu as pltpu
from jax.experimental.pallas import tpu_sc as plsc

VOCAB = 100000
DIM = 64
NVALS = 81920

NC = 2
NS = 16
NW = NC * NS

B_PER_W = NVALS // NW
CHUNK = 640
CPW = B_PER_W // CHUNK
NBUF = 3


@functools.partial(
    pl.kernel,
    out_type=(
        jax.ShapeDtypeStruct((NVALS, DIM), jnp.float32),
        jax.ShapeDtypeStruct((NVALS, DIM), jnp.float32),
    ),
    mesh=plsc.VectorSubcoreMesh(core_axis_name="c", subcore_axis_name="s"),
    compiler_params=pltpu.CompilerParams(use_tc_tiling_on_sc=False),
    scratch_types=[
        pltpu.VMEM((B_PER_W,), jnp.int32),
        pltpu.VMEM((B_PER_W,), jnp.int32),
        pltpu.VMEM((NBUF, CHUNK, DIM), jnp.float32),
        [pltpu.SemaphoreType.DMA] * NBUF,
        [pltpu.SemaphoreType.DMA] * NBUF,
    ],
)
def _lookup2(v1_hbm, v2_hbm, t1_hbm, t2_hbm, o1_hbm, o2_hbm,
             idx1_v, idx2_v, bufs, gsems, ssems):
    wid = lax.axis_index("s") * NC + lax.axis_index("c")
    base = wid * B_PER_W

    pltpu.sync_copy(v1_hbm.at[pl.ds(base, B_PER_W)], idx1_v)
    pltpu.sync_copy(v2_hbm.at[pl.ds(base, B_PER_W)], idx2_v)

    chunks = [(t1_hbm, idx1_v, o1_hbm, c * CHUNK) for c in range(CPW)]
    chunks += [(t2_hbm, idx2_v, o2_hbm, c * CHUNK) for c in range(CPW)]
    n = len(chunks)

    def fire_gather(i):
        tbl, idx, _, off = chunks[i]
        return pltpu.async_copy(
            tbl.at[idx.at[pl.ds(off, CHUNK)]], bufs.at[i % NBUF],
            gsems[i % NBUF])

    def fire_scatter(i):
        _, _, out, off = chunks[i]
        return pltpu.async_copy(
            bufs.at[i % NBUF], out.at[pl.ds(base + off, CHUNK)],
            ssems[i % NBUF])

    gh = [None] * n
    sh = [None] * n
    gh[0] = fire_gather(0)
    gh[1] = fire_gather(1)
    for i in range(n):
        gh[i].wait()
        sh[i] = fire_scatter(i)
        if i + 2 < n:
            if i >= 1:
                sh[i - 1].wait()
            gh[i + 2] = fire_gather(i + 2)
    for i in range(n - NBUF, n):
        sh[i].wait()


def kernel(values_f1, values_f2, table_t1, table_t2):
    return _lookup2(values_f1, values_f2, table_t1, table_t2)


# speedup vs baseline: 1.1801x; 1.0702x over previous
import functools

import jax
import jax.numpy as jnp
from jax import lax
from jax.experimental import pallas as pl
from jax.experimental.pallas import tpu as pltpu
from jax.experimental.pallas import tpu_sc as plsc

VOCAB = 100000
DIM = 64
NVALS = 81920

NS = 16

B_PER_W = NVALS // NS
CHUNK = 640
CPW = B_PER_W // CHUNK
NBUF = 3


@functools.partial(
    pl.kernel,
    out_type=jax.ShapeDtypeStruct((NVALS, DIM), jnp.float32),
    mesh=plsc.VectorSubcoreMesh(
        core_axis_name="c", subcore_axis_name="s", num_cores=1),
    compiler_params=pltpu.CompilerParams(use_tc_tiling_on_sc=False),
    scratch_types=[
        pltpu.VMEM((B_PER_W,), jnp.int32),
        pltpu.VMEM((NBUF, CHUNK, DIM), jnp.float32),
        [pltpu.SemaphoreType.DMA] * NBUF,
        [pltpu.SemaphoreType.DMA] * NBUF,
    ],
)
def _lookup1(vals_hbm, table_hbm, out_hbm, idx_v, bufs, gsems, ssems):
    wid = lax.axis_index("s")
    base = wid * B_PER_W

    pltpu.sync_copy(vals_hbm.at[pl.ds(base, B_PER_W)], idx_v)

    def fire_gather(i):
        return pltpu.async_copy(
            table_hbm.at[idx_v.at[pl.ds(i * CHUNK, CHUNK)]],
            bufs.at[i % NBUF], gsems[i % NBUF])

    def fire_scatter(i):
        return pltpu.async_copy(
            bufs.at[i % NBUF], out_hbm.at[pl.ds(base + i * CHUNK, CHUNK)],
            ssems[i % NBUF])

    gh = [None] * CPW
    sh = [None] * CPW
    gh[0] = fire_gather(0)
    gh[1] = fire_gather(1)
    for i in range(CPW):
        gh[i].wait()
        sh[i] = fire_scatter(i)
        if i + 2 < CPW:
            if i >= 1:
                sh[i - 1].wait()
            gh[i + 2] = fire_gather(i + 2)
    for i in range(CPW - NBUF, CPW):
        sh[i].wait()


def kernel(values_f1, values_f2, table_t1, table_t2):
    return (_lookup1(values_f1, table_t1), _lookup1(values_f2, table_t2))
